# baseline (speedup 1.0000x reference)
import functools

import jax
import jax.numpy as jnp
from jax.experimental import pallas as pl
from jax.experimental.pallas import tpu as pltpu

_D = 1024
_H = 4096
_E = 8
_K = 2
_N = 2048
_B = 512
_G = (_K * _N + (_B - 1) * _E) // _B
_HB = 1024
_NH = _H // _HB


def _router_body(x_ref, gw_ref, r_ref, w_ref, cnt_ref):
    x = x_ref[...]
    logits = jnp.dot(x, gw_ref[...], preferred_element_type=jnp.float32)
    mx = jnp.max(logits, axis=-1, keepdims=True)
    p = jnp.exp(logits - mx)
    scores = p / jnp.sum(p, axis=-1, keepdims=True)

    eidx = jax.lax.broadcasted_iota(jnp.int32, (_N, _E), 1)
    m1 = jnp.max(scores, axis=-1, keepdims=True)
    i1 = jnp.min(jnp.where(scores == m1, eidx, _E), axis=-1, keepdims=True)
    oh1 = (eidx == i1)
    s2 = jnp.where(oh1, -jnp.inf, scores)
    m2 = jnp.max(s2, axis=-1, keepdims=True)
    i2 = jnp.min(jnp.where(s2 == m2, eidx, _E), axis=-1, keepdims=True)
    oh2 = (eidx == i2)

    denom = m1 + m2
    wd = (m1 * oh1.astype(jnp.float32) + m2 * oh2.astype(jnp.float32)) / denom
    assigned = (oh1 | oh2).astype(jnp.float32)

    ri = jax.lax.broadcasted_iota(jnp.int32, (_N, _N), 0)
    ci = jax.lax.broadcasted_iota(jnp.int32, (_N, _N), 1)
    tri = (ci < ri).astype(jnp.float32)
    rank = jnp.dot(tri, assigned, preferred_element_type=jnp.float32)

    r_ref[...] = jnp.where(assigned > 0, rank, -1.0)
    w_ref[...] = wd
    counts = jnp.sum(assigned, axis=0, keepdims=True)
    cnt_ref[...] = jnp.broadcast_to(counts, (8, _E))


def _ffn_body(md_sref, x_ref, r_ref, rt_ref, w_ref, w1_ref, b1_ref,
              w2_ref, b2_ref, out_ref, xg_ref, oc_ref):
    b = pl.program_id(0)
    hb = pl.program_id(1)

    @pl.when((b == 0) & (hb == 0))
    def _init():
        out_ref[...] = jnp.zeros_like(out_ref)

    e = md_sref[b]
    nblk = md_sref[_G + _E]

    @pl.when(b < nblk)
    def _work():
        off_e = md_sref[_G + e]
        basef = (b * _B - off_e).astype(jnp.float32)

        @pl.when(hb == 0)
        def _gather():
            srow = jax.lax.broadcasted_iota(jnp.int32, (_E, 1), 0)
            rt_e = jnp.sum(jnp.where(srow == e, rt_ref[...], 0.0), axis=0,
                           keepdims=True)
            rows = jax.lax.broadcasted_iota(
                jnp.int32, (_B, _N), 0).astype(jnp.float32) + basef
            dmat = (rows == rt_e).astype(jnp.float32)
            xg_ref[...] = jnp.dot(
                dmat, x_ref[...], preferred_element_type=jnp.float32)

        h = jnp.dot(xg_ref[...].astype(jnp.bfloat16), w1_ref[0],
                    preferred_element_type=jnp.float32)
        h = h + b1_ref[0]
        h = 0.5 * h * (1.0 + jax.lax.erf(h * 0.7071067811865476))
        oc = jnp.dot(h.astype(jnp.bfloat16), w2_ref[0],
                     preferred_element_type=jnp.float32)

        @pl.when(hb == 0)
        def _oc_first():
            oc_ref[...] = oc + b2_ref[0]

        @pl.when(hb > 0)
        def _oc_rest():
            oc_ref[...] += oc

        @pl.when(hb == _NH - 1)
        def _scatter():
            lane_e = jax.lax.broadcasted_iota(jnp.int32, (_N, _E), 1)
            sel = lane_e == e
            rcol = jnp.sum(jnp.where(sel, r_ref[...], 0.0), axis=1,
                           keepdims=True)
            wcol = jnp.sum(jnp.where(sel, w_ref[...], 0.0), axis=1,
                           keepdims=True)
            cols = jax.lax.broadcasted_iota(
                jnp.int32, (_N, _B), 1).astype(jnp.float32) + basef
            dwt = jnp.where(cols == rcol, wcol, 0.0)
            out_ref[...] += jnp.dot(dwt, oc_ref[...],
                                    preferred_element_type=jnp.float32)


@functools.partial(jax.jit, static_argnames=("interpret",))
def _moe(x, gate_w, w1, b1, w2, b2, interpret=False):
    B0, T, D = x.shape
    xf = x.reshape(-1, D)

    r, wd, cnt8 = pl.pallas_call(
        _router_body,
        out_shape=(
            jax.ShapeDtypeStruct((_N, _E), jnp.float32),
            jax.ShapeDtypeStruct((_N, _E), jnp.float32),
            jax.ShapeDtypeStruct((8, _E), jnp.float32),
        ),
        interpret=interpret,
    )(xf, gate_w)

    counts = cnt8[0].astype(jnp.int32)
    rt = r.T
    b1r = b1.reshape(_E, 1, _H)
    b2r = b2.reshape(_E, 1, _D)

    pc = (counts + _B - 1) // _B
    off_blk = jnp.cumsum(pc) - pc
    nblk = jnp.sum(pc)
    bidx = jnp.arange(_G, dtype=jnp.int32)
    ends = off_blk + pc
    eid = jnp.minimum(
        jnp.sum((bidx[:, None] >= ends[None, :]).astype(jnp.int32), axis=1),
        _E - 1)
    md = jnp.concatenate(
        [eid, off_blk * _B, nblk.reshape(1)]).astype(jnp.int32)

    grid_spec = pltpu.PrefetchScalarGridSpec(
        num_scalar_prefetch=1,
        grid=(_G, _NH),
        in_specs=[
            pl.BlockSpec((_N, _D), lambda b, h, m: (0, 0)),
            pl.BlockSpec((_N, _E), lambda b, h, m: (0, 0)),
            pl.BlockSpec((_E, _N), lambda b, h, m: (0, 0)),
            pl.BlockSpec((_N, _E), lambda b, h, m: (0, 0)),
            pl.BlockSpec((1, _D, _HB), lambda b, h, m: (m[b], 0, h)),
            pl.BlockSpec((1, 1, _HB), lambda b, h, m: (m[b], 0, h)),
            pl.BlockSpec((1, _HB, _D), lambda b, h, m: (m[b], h, 0)),
            pl.BlockSpec((1, 1, _D), lambda b, h, m: (m[b], 0, 0)),
        ],
        out_specs=pl.BlockSpec((_N, _D), lambda b, h, m: (0, 0)),
        scratch_shapes=[
            pltpu.VMEM((_B, _D), jnp.float32),
            pltpu.VMEM((_B, _D), jnp.float32),
        ],
    )

    out = pl.pallas_call(
        _ffn_body,
        grid_spec=grid_spec,
        out_shape=jax.ShapeDtypeStruct((_N, _D), jnp.float32),
        interpret=interpret,
    )(md, xf, r, rt, wd, w1.astype(jnp.bfloat16), b1r,
      w2.astype(jnp.bfloat16), b2r)

    return out.reshape(B0, T, D)


def kernel(x, gate_w, w1, b1, w2, b2):
    return _moe(x, gate_w, w1, b1, w2, b2)


# speedup vs baseline: 1.3918x; 1.3918x over previous
import functools

import jax
import jax.numpy as jnp
from jax.experimental import pallas as pl
from jax.experimental.pallas import tpu as pltpu

_D = 1024
_H = 4096
_E = 8
_K = 2
_N = 2048
_B = 512
_G = (_K * _N + (_B - 1) * _E) // _B
_HB = 1024
_NH = _H // _HB


def _router_body(x_ref, gw_ref, r_ref, w_ref, cnt_ref):
    x = x_ref[...]
    logits = jnp.dot(x, gw_ref[...], preferred_element_type=jnp.float32)
    mx = jnp.max(logits, axis=-1, keepdims=True)
    p = jnp.exp(logits - mx)
    scores = p / jnp.sum(p, axis=-1, keepdims=True)

    eidx = jax.lax.broadcasted_iota(jnp.int32, (_N, _E), 1)
    m1 = jnp.max(scores, axis=-1, keepdims=True)
    i1 = jnp.min(jnp.where(scores == m1, eidx, _E), axis=-1, keepdims=True)
    oh1 = (eidx == i1)
    s2 = jnp.where(oh1, -jnp.inf, scores)
    m2 = jnp.max(s2, axis=-1, keepdims=True)
    i2 = jnp.min(jnp.where(s2 == m2, eidx, _E), axis=-1, keepdims=True)
    oh2 = (eidx == i2)

    denom = m1 + m2
    wd = (m1 * oh1.astype(jnp.float32) + m2 * oh2.astype(jnp.float32)) / denom
    assigned = (oh1 | oh2).astype(jnp.float32)

    ri = jax.lax.broadcasted_iota(jnp.int32, (_N, _N), 0)
    ci = jax.lax.broadcasted_iota(jnp.int32, (_N, _N), 1)
    tri = (ci < ri).astype(jnp.float32)
    rank = jnp.dot(tri, assigned, preferred_element_type=jnp.float32)

    r_ref[...] = jnp.where(assigned > 0, rank, -1.0)
    w_ref[...] = wd
    counts = jnp.sum(assigned, axis=0, keepdims=True)
    cnt_ref[...] = jnp.broadcast_to(counts, (8, _E))


def _ffn_body(md_sref, x_ref, r_ref, rt_ref, w_ref, w1_ref, b1_ref,
              w2_ref, b2_ref, out_ref, xg_ref, oc_ref):
    b = pl.program_id(0)
    hb = pl.program_id(1)

    @pl.when((b == 0) & (hb == 0))
    def _init():
        out_ref[...] = jnp.zeros_like(out_ref)

    e = md_sref[b]
    nblk = md_sref[_G + _E]

    @pl.when(b < nblk)
    def _work():
        off_e = md_sref[_G + e]
        basef = (b * _B - off_e).astype(jnp.float32)

        @pl.when(hb == 0)
        def _gather():
            srow = jax.lax.broadcasted_iota(jnp.int32, (_E, 1), 0)
            rt_e = jnp.sum(jnp.where(srow == e, rt_ref[...], 0.0), axis=0,
                           keepdims=True)
            rows = jax.lax.broadcasted_iota(
                jnp.int32, (_B, _N), 0).astype(jnp.float32) + basef
            dmat = (rows == rt_e).astype(jnp.float32)
            xg_ref[...] = jnp.dot(
                dmat, x_ref[...], preferred_element_type=jnp.float32)

        h = jnp.dot(xg_ref[...], w1_ref[0], preferred_element_type=jnp.float32)
        h = h + b1_ref[0]
        h = 0.5 * h * (1.0 + jax.lax.erf(h * 0.7071067811865476))
        oc = jnp.dot(h, w2_ref[0], preferred_element_type=jnp.float32)

        @pl.when(hb == 0)
        def _oc_first():
            oc_ref[...] = oc + b2_ref[0]

        @pl.when(hb > 0)
        def _oc_rest():
            oc_ref[...] += oc

        @pl.when(hb == _NH - 1)
        def _scatter():
            lane_e = jax.lax.broadcasted_iota(jnp.int32, (_N, _E), 1)
            sel = lane_e == e
            rcol = jnp.sum(jnp.where(sel, r_ref[...], 0.0), axis=1,
                           keepdims=True)
            wcol = jnp.sum(jnp.where(sel, w_ref[...], 0.0), axis=1,
                           keepdims=True)
            cols = jax.lax.broadcasted_iota(
                jnp.int32, (_N, _B), 1).astype(jnp.float32) + basef
            dwt = jnp.where(cols == rcol, wcol, 0.0)
            out_ref[...] += jnp.dot(dwt, oc_ref[...],
                                    preferred_element_type=jnp.float32)


@functools.partial(jax.jit, static_argnames=("interpret",))
def _moe(x, gate_w, w1, b1, w2, b2, interpret=False):
    B0, T, D = x.shape
    xf = x.reshape(-1, D)

    r, wd, cnt8 = pl.pallas_call(
        _router_body,
        out_shape=(
            jax.ShapeDtypeStruct((_N, _E), jnp.float32),
            jax.ShapeDtypeStruct((_N, _E), jnp.float32),
            jax.ShapeDtypeStruct((8, _E), jnp.float32),
        ),
        interpret=interpret,
    )(xf, gate_w)

    counts = cnt8[0].astype(jnp.int32)
    rt = r.T
    b1r = b1.reshape(_E, 1, _H)
    b2r = b2.reshape(_E, 1, _D)

    pc = (counts + _B - 1) // _B
    off_blk = jnp.cumsum(pc) - pc
    nblk = jnp.sum(pc)
    bidx = jnp.arange(_G, dtype=jnp.int32)
    ends = off_blk + pc
    eid = jnp.minimum(
        jnp.sum((bidx[:, None] >= ends[None, :]).astype(jnp.int32), axis=1),
        _E - 1)
    md = jnp.concatenate(
        [eid, off_blk * _B, nblk.reshape(1)]).astype(jnp.int32)

    grid_spec = pltpu.PrefetchScalarGridSpec(
        num_scalar_prefetch=1,
        grid=(_G, _NH),
        in_specs=[
            pl.BlockSpec((_N, _D), lambda b, h, m: (0, 0)),
            pl.BlockSpec((_N, _E), lambda b, h, m: (0, 0)),
            pl.BlockSpec((_E, _N), lambda b, h, m: (0, 0)),
            pl.BlockSpec((_N, _E), lambda b, h, m: (0, 0)),
            pl.BlockSpec((1, _D, _HB), lambda b, h, m: (m[b], 0, h)),
            pl.BlockSpec((1, 1, _HB), lambda b, h, m: (m[b], 0, h)),
            pl.BlockSpec((1, _HB, _D), lambda b, h, m: (m[b], h, 0)),
            pl.BlockSpec((1, 1, _D), lambda b, h, m: (m[b], 0, 0)),
        ],
        out_specs=pl.BlockSpec((_N, _D), lambda b, h, m: (0, 0)),
        scratch_shapes=[
            pltpu.VMEM((_B, _D), jnp.float32),
            pltpu.VMEM((_B, _D), jnp.float32),
        ],
    )

    out = pl.pallas_call(
        _ffn_body,
        grid_spec=grid_spec,
        out_shape=jax.ShapeDtypeStruct((_N, _D), jnp.float32),
        interpret=interpret,
    )(md, xf, r, rt, wd, w1, b1r, w2, b2r)

    return out.reshape(B0, T, D)


def kernel(x, gate_w, w1, b1, w2, b2):
    return _moe(x, gate_w, w1, b1, w2, b2)
